# baseline (speedup 1.0000x reference)
import functools
import math

import jax
import jax.numpy as jnp
from jax import lax
from jax.experimental import pallas as pl
from jax.experimental.pallas import tpu as pltpu
from jax.experimental.pallas import tpu_sc as plsc

_D = 64
_SCALE = math.sqrt(_D)
_NC = 2
_NS = 16
_NW = _NC * _NS
_LANES = 16
_CHUNK = 640
_SUB = 128


@functools.partial(jax.jit, static_argnames=("n_per_w",))
def _embed(x_flat, table2, n_per_w):
    n = x_flat.shape[0]
    mesh = plsc.VectorSubcoreMesh(core_axis_name="c", subcore_axis_name="s")

    @functools.partial(
        pl.kernel,
        mesh=mesh,
        out_type=jax.ShapeDtypeStruct((n // 2, 2 * _D), jnp.float32),
        scratch_types=[
            pltpu.VMEM((_CHUNK,), jnp.int32),
            pltpu.VMEM((_CHUNK,), jnp.int32),
            pltpu.VMEM((_CHUNK, 2 * _D), jnp.float32),
            pltpu.VMEM((_CHUNK // 2, 2 * _D), jnp.float32),
            pltpu.SemaphoreType.DMA,
        ],
    )
    def k(x_hbm, table_hbm, out_hbm, idx_v, u_v, rows_v, out_v, sem):
        wid = lax.axis_index("s") * _NC + lax.axis_index("c")
        base = wid * n_per_w

        def chunk_body(ci, carry):
            off = pl.multiple_of(base + ci * _CHUNK, _CHUNK)
            pltpu.sync_copy(x_hbm.at[pl.ds(off, _CHUNK)], idx_v)

            def shift_body(g, c2):
                sl = pl.ds(pl.multiple_of(g * _LANES, _LANES), _LANES)
                u_v[sl] = lax.shift_right_logical(idx_v[sl], 1)
                return c2

            lax.fori_loop(0, _CHUNK // _LANES, shift_body, 0, unroll=4)

            copies = []
            for j in range(_CHUNK // _SUB):
                copies.append(
                    pltpu.async_copy(
                        table_hbm.at[u_v.at[pl.ds(j * _SUB, _SUB)]],
                        rows_v.at[pl.ds(j * _SUB, _SUB), :],
                        sem,
                    )
                )
            for c in copies:
                c.wait()

            def sel_body(g, c2):
                i0 = pl.multiple_of(g * _LANES, _LANES)
                colvec = (idx_v[pl.ds(i0, _LANES)] & 1) * _D
                for e in range(_LANES):
                    col = colvec[e]
                    r = i0 + e
                    ro = lax.shift_right_logical(r, 1)
                    co = (e & 1) * _D
                    for m in range(_D // _LANES):
                        src = pl.ds(col + m * _LANES, _LANES)
                        dst = pl.ds(co + m * _LANES, _LANES)
                        out_v[ro, dst] = rows_v[r, src] + _SCALE
                return c2

            lax.fori_loop(0, _CHUNK // _LANES, sel_body, 0)

            off2 = pl.multiple_of(off // 2, _CHUNK // 2)
            pltpu.sync_copy(out_v, out_hbm.at[pl.ds(off2, _CHUNK // 2)])
            return carry

        lax.fori_loop(0, n_per_w // _CHUNK, chunk_body, 0)

    return k(x_flat, table2)


def kernel(x, table):
    b, l = x.shape
    n = b * l
    n_per_w = n // _NW
    x_flat = x.reshape(n).astype(jnp.int32)
    table2 = table.reshape(-1, 2 * _D)
    out = _embed(x_flat, table2, n_per_w)
    return out.reshape(b, l, _D)


# speedup vs baseline: 1.0928x; 1.0928x over previous
import functools
import math

import jax
import jax.numpy as jnp
from jax import lax
from jax.experimental import pallas as pl
from jax.experimental.pallas import tpu as pltpu
from jax.experimental.pallas import tpu_sc as plsc

_D = 64
_SCALE = math.sqrt(_D)
_NC = 2
_NS = 16
_NW = _NC * _NS
_LANES = 16
_CHUNK = 128


@functools.partial(jax.jit, static_argnames=("n_per_w",))
def _embed(x_flat, table, n_per_w):
    n = x_flat.shape[0]
    mesh = plsc.VectorSubcoreMesh(core_axis_name="c", subcore_axis_name="s")

    @functools.partial(
        pl.kernel,
        mesh=mesh,
        compiler_params=pltpu.CompilerParams(
            use_tc_tiling_on_sc=False, needs_layout_passes=False),
        out_type=jax.ShapeDtypeStruct((n, _D), jnp.float32),
        scratch_types=[
            pltpu.VMEM((_CHUNK,), jnp.int32),
            pltpu.VMEM((_CHUNK, _D), jnp.float32),
            pltpu.SemaphoreType.DMA,
        ],
    )
    def k(x_hbm, table_hbm, out_hbm, idx_v, rows_v, sem):
        wid = lax.axis_index("s") * _NC + lax.axis_index("c")
        base = wid * n_per_w

        def chunk_body(ci, carry):
            off = pl.multiple_of(base + ci * _CHUNK, _CHUNK)
            pltpu.sync_copy(x_hbm.at[pl.ds(off, _CHUNK)], idx_v)
            pltpu.async_copy(table_hbm.at[idx_v], rows_v, sem).wait()

            def row_body(r, c2):
                for j in range(_D // _LANES):
                    sl = pl.ds(j * _LANES, _LANES)
                    rows_v[r, sl] = rows_v[r, sl] + _SCALE
                return c2

            lax.fori_loop(0, _CHUNK, row_body, 0, unroll=2)
            pltpu.sync_copy(rows_v, out_hbm.at[pl.ds(off, _CHUNK)])
            return carry

        lax.fori_loop(0, n_per_w // _CHUNK, chunk_body, 0)

    return k(x_flat, table)


def kernel(x, table):
    b, l = x.shape
    n = b * l
    n_per_w = n // _NW
    x_flat = x.reshape(n).astype(jnp.int32)
    out = _embed(x_flat, table, n_per_w)
    return out.reshape(b, l, _D)


# speedup vs baseline: 1.1880x; 1.0871x over previous
import functools
import math

import jax
import jax.numpy as jnp
from jax import lax
from jax.experimental import pallas as pl
from jax.experimental.pallas import tpu as pltpu
from jax.experimental.pallas import tpu_sc as plsc

_D = 64
_SCALE = math.sqrt(_D)
_NC = 2
_NS = 16
_NW = _NC * _NS
_LANES = 16
_CHUNK = 640
_SUB = 128


@functools.partial(jax.jit, static_argnames=("n_per_w",))
def _embed(x_flat, table, n_per_w):
    n = x_flat.shape[0]
    n_chunks = n_per_w // _CHUNK
    mesh = plsc.VectorSubcoreMesh(core_axis_name="c", subcore_axis_name="s")

    @functools.partial(
        pl.kernel,
        mesh=mesh,
        compiler_params=pltpu.CompilerParams(
            use_tc_tiling_on_sc=False, needs_layout_passes=False),
        out_type=jax.ShapeDtypeStruct((n, _D), jnp.float32),
        scratch_types=[
            pltpu.VMEM((n_per_w,), jnp.int32),
            pltpu.VMEM((_CHUNK, _D), jnp.float32),
            pltpu.VMEM((_CHUNK, _D), jnp.float32),
            pltpu.SemaphoreType.DMA,
            pltpu.SemaphoreType.DMA,
            pltpu.SemaphoreType.DMA,
            pltpu.SemaphoreType.DMA,
        ],
    )
    def k(x_hbm, table_hbm, out_hbm, idx_v, rows0, rows1, g0, g1, s0, s1):
        wid = lax.axis_index("s") * _NC + lax.axis_index("c")
        base = pl.multiple_of(wid * n_per_w, _CHUNK)
        pltpu.sync_copy(x_hbm.at[pl.ds(base, n_per_w)], idx_v)

        rows = (rows0, rows1)
        gsem = (g0, g1)
        ssem = (s0, s1)

        def fire_gathers(ci, buf):
            descs = []
            for j in range(_CHUNK // _SUB):
                o = ci * _CHUNK + j * _SUB
                descs.append(
                    pltpu.async_copy(
                        table_hbm.at[idx_v.at[pl.ds(o, _SUB)]],
                        rows[buf].at[pl.ds(j * _SUB, _SUB), :],
                        gsem[buf],
                    )
                )
            return descs

        def add_pass(buf):
            def row_body(r, c2):
                for j in range(_D // _LANES):
                    sl = pl.ds(j * _LANES, _LANES)
                    rows[buf][r, sl] = rows[buf][r, sl] + _SCALE
                return c2

            lax.fori_loop(0, _CHUNK, row_body, 0, unroll=4)

        gd = {}
        sd = {}
        for ci in range(n_chunks + 1):
            if ci < n_chunks:
                buf = ci & 1
                if ci >= 2:
                    sd[ci - 2].wait()
                gd[ci] = fire_gathers(ci, buf)
            if ci >= 1:
                pbuf = (ci - 1) & 1
                for d in gd[ci - 1]:
                    d.wait()
                add_pass(pbuf)
                off = pl.multiple_of(base + (ci - 1) * _CHUNK, _CHUNK)
                sd[ci - 1] = pltpu.async_copy(
                    rows[pbuf], out_hbm.at[pl.ds(off, _CHUNK)], ssem[pbuf]
                )
        sd[n_chunks - 2].wait()
        sd[n_chunks - 1].wait()

    return k(x_flat, table)


def kernel(x, table):
    b, l = x.shape
    n = b * l
    n_per_w = n // _NW
    x_flat = x.reshape(n).astype(jnp.int32)
    out = _embed(x_flat, table, n_per_w)
    return out.reshape(b, l, _D)
